# baseline (speedup 1.0000x reference)
import jax
import jax.numpy as jnp
from jax import lax
from jax.experimental import pallas as pl
from jax.experimental.pallas import tpu as pltpu
from jax.experimental.pallas import tpu_sc as plsc

BATCH = 16384
HIST = 200
EMB = 32
N = BATCH * HIST
ROWS = 100001
WINDOW = 128
K = 4
NSUB = 16
ROWS_PER_SUB = ROWS // NSUB


def _sc_gather(tab16_hbm, idx_hbm, out16_hbm, spm, sem):
    s = lax.axis_index("s")
    pltpu.sync_copy(
        tab16_hbm.at[pl.ds(s * ROWS_PER_SUB, ROWS_PER_SUB)],
        spm.at[pl.ds(s * ROWS_PER_SUB, ROWS_PER_SUB)],
    )

    @pl.when(s == 0)
    def _():
        pltpu.sync_copy(
            tab16_hbm.at[pl.ds(NSUB * ROWS_PER_SUB, ROWS - NSUB * ROWS_PER_SUB)],
            spm.at[pl.ds(NSUB * ROWS_PER_SUB, ROWS - NSUB * ROWS_PER_SUB)],
        )

    plsc.subcore_barrier()

    def body(i_vmem, o_vmem):
        copies = [
            pltpu.async_copy(
                spm.at[i_vmem.at[j]],
                o_vmem.at[pl.ds(j * WINDOW, WINDOW)],
                sem,
            )
            for j in range(K)
        ]
        for c in copies:
            c.wait()

    pltpu.emit_pipeline(
        body,
        grid=(N // (WINDOW * K),),
        in_specs=[pl.BlockSpec((K, WINDOW), index_map=lambda i: (i, 0))],
        out_specs=[pl.BlockSpec((K * WINDOW, EMB), index_map=lambda i: (i, 0))],
        core_axis_name=("c", "s"),
        dimension_semantics=(pltpu.PARALLEL,),
    )(idx_hbm, out16_hbm)


def _tc_upcast(x_ref, o_ref):
    o_ref[...] = x_ref[...].astype(jnp.float32)


@jax.jit
def kernel(prev_action, action_emb_weight):
    tab16 = action_emb_weight.astype(jnp.bfloat16)
    idx = prev_action.reshape(N // WINDOW, WINDOW).astype(jnp.int32)
    mesh = plsc.VectorSubcoreMesh(core_axis_name="c", subcore_axis_name="s")
    out16 = pl.kernel(
        _sc_gather,
        out_type=jax.ShapeDtypeStruct((N, EMB), jnp.bfloat16),
        mesh=mesh,
        scratch_types=[
            pltpu.VMEM_SHARED((ROWS, EMB), jnp.bfloat16),
            pltpu.SemaphoreType.DMA,
        ],
        compiler_params=pltpu.CompilerParams(use_tc_tiling_on_sc=False),
    )(tab16, idx)

    flat16 = out16.reshape(N * EMB // 4096, 4096)
    BLK = 256
    out = pl.pallas_call(
        _tc_upcast,
        out_shape=jax.ShapeDtypeStruct(flat16.shape, jnp.float32),
        grid=(flat16.shape[0] // BLK,),
        in_specs=[pl.BlockSpec((BLK, 4096), lambda i: (i, 0))],
        out_specs=pl.BlockSpec((BLK, 4096), lambda i: (i, 0)),
    )(flat16)
    return out.reshape(BATCH, HIST, EMB)


# speedup vs baseline: 1.4410x; 1.4410x over previous
import jax
import jax.numpy as jnp
from jax.experimental import pallas as pl
from jax.experimental.pallas import tpu as pltpu
from jax.experimental.pallas import tpu_sc as plsc

BATCH = 16384
HIST = 200
EMB = 32
N = BATCH * HIST
WINDOW = 128
K = 8


def _sc_gather(table_hbm, idx_hbm, out_hbm, sem):
    def body(i_vmem, o_vmem):
        copies = [
            pltpu.async_copy(
                table_hbm.at[i_vmem.at[j]],
                o_vmem.at[pl.ds(j * WINDOW, WINDOW)],
                sem,
            )
            for j in range(K)
        ]
        for c in copies:
            c.wait()

    pltpu.emit_pipeline(
        body,
        grid=(N // (WINDOW * K),),
        in_specs=[pl.BlockSpec((K, WINDOW), index_map=lambda i: (i, 0))],
        out_specs=[pl.BlockSpec((K * WINDOW, EMB), index_map=lambda i: (i, 0))],
        core_axis_name=("c", "s"),
        dimension_semantics=(pltpu.PARALLEL,),
    )(idx_hbm, out_hbm)


@jax.jit
def kernel(prev_action, action_emb_weight):
    idx = prev_action.reshape(N // WINDOW, WINDOW).astype(jnp.int32)
    mesh = plsc.VectorSubcoreMesh(core_axis_name="c", subcore_axis_name="s")
    out = pl.kernel(
        _sc_gather,
        out_type=jax.ShapeDtypeStruct((N, EMB), jnp.float32),
        mesh=mesh,
        scratch_types=[pltpu.SemaphoreType.DMA],
        compiler_params=pltpu.CompilerParams(use_tc_tiling_on_sc=False),
    )(action_emb_weight, idx)
    return out.reshape(BATCH, HIST, EMB)
